# baseline (speedup 1.0000x reference)
import functools

import jax
import jax.numpy as jnp
from jax import lax
from jax.experimental import pallas as pl
from jax.experimental.pallas import tpu as pltpu
from jax.experimental.pallas import tpu_sc as plsc

_NUM_CORES = 2
_NUM_SUBCORES = 16
_LANES = 16


def _sc_gather_rows(x_flat, w, *, interpret=False):
    D, V = w.shape
    T = x_flat.shape[0]
    nw = _NUM_CORES * _NUM_SUBCORES
    rows_per_w = D // nw
    assert D % nw == 0 and T % _LANES == 0 and V % 8 == 0

    mesh = plsc.VectorSubcoreMesh(
        core_axis_name="c", subcore_axis_name="s",
        num_cores=_NUM_CORES, num_subcores=_NUM_SUBCORES)

    @functools.partial(
        pl.kernel,
        out_type=jax.ShapeDtypeStruct((D, T), jnp.float32),
        mesh=mesh,
        scratch_types=[
            pltpu.VMEM((T,), jnp.int32),
            pltpu.VMEM((V,), jnp.float32),
            pltpu.VMEM((T,), jnp.float32),
        ],
        compiler_params=pltpu.CompilerParams(needs_layout_passes=False),
        interpret=interpret,
    )
    def sc_embed(x_hbm, w_hbm, out_hbm, idx_v, row_v, val_v):
        wid = lax.axis_index("s") * _NUM_CORES + lax.axis_index("c")
        d0 = wid * rows_per_w
        pltpu.sync_copy(x_hbm, idx_v)

        @pl.loop(0, rows_per_w)
        def _row(r):
            d = d0 + r
            pltpu.sync_copy(w_hbm.at[d], row_v)

            @pl.loop(0, T // _LANES)
            def _chunk(i):
                o = pl.multiple_of(i * _LANES, _LANES)
                ids = idx_v[pl.ds(o, _LANES)]
                val_v[pl.ds(o, _LANES)] = plsc.load_gather(row_v, [ids])

            pltpu.sync_copy(val_v, out_hbm.at[d])

    return sc_embed(x_flat, w)


def _tc_transpose(out_t, *, interpret=False):
    D, T = out_t.shape
    tblk = 512
    assert T % tblk == 0

    def body(in_ref, out_ref):
        out_ref[...] = in_ref[...].T

    return pl.pallas_call(
        body,
        grid=(T // tblk,),
        in_specs=[pl.BlockSpec((D, tblk), lambda i: (0, i))],
        out_specs=pl.BlockSpec((tblk, D), lambda i: (i, 0)),
        out_shape=jax.ShapeDtypeStruct((T, D), jnp.float32),
        interpret=interpret,
    )(out_t)


def kernel(x, W_E):
    B, S = x.shape
    D, V = W_E.shape
    x_flat = x.reshape(B * S).astype(jnp.int32)
    out_t = _sc_gather_rows(x_flat, W_E)
    out = _tc_transpose(out_t)
    return out.reshape(B, S, D)


# speedup vs baseline: 12.9162x; 12.9162x over previous
import functools

import jax
import jax.numpy as jnp
from jax import lax
from jax.experimental import pallas as pl
from jax.experimental.pallas import tpu as pltpu
from jax.experimental.pallas import tpu_sc as plsc

_NUM_CORES = 2
_NUM_SUBCORES = 16


def _sc_row_gather(x_flat, w_t):
    V, D = w_t.shape
    T = x_flat.shape[0]
    nw = _NUM_CORES * _NUM_SUBCORES
    b_per_w = T // nw
    ch = 64
    n_ch = b_per_w // ch
    assert T % (8 * nw) == 0 and b_per_w % ch == 0

    mesh = plsc.VectorSubcoreMesh(
        core_axis_name="c", subcore_axis_name="s",
        num_cores=_NUM_CORES, num_subcores=_NUM_SUBCORES)

    @functools.partial(
        pl.kernel,
        out_type=jax.ShapeDtypeStruct((T, D), jnp.float32),
        mesh=mesh,
        scratch_types=[
            pltpu.VMEM((b_per_w,), jnp.int32),
            pltpu.VMEM((2, ch, D), jnp.float32),
            pltpu.SemaphoreType.DMA,
            pltpu.SemaphoreType.DMA,
            pltpu.SemaphoreType.DMA,
        ],
        compiler_params=pltpu.CompilerParams(needs_layout_passes=False),
    )
    def sc_gather(x_hbm, w_hbm, out_hbm, idx_v, rows_v, g_sem, w_sem0, w_sem1):
        wid = lax.axis_index("s") * _NUM_CORES + lax.axis_index("c")
        base = wid * b_per_w
        pltpu.sync_copy(x_hbm.at[pl.ds(base, b_per_w)], idx_v)

        w_sems = (w_sem0, w_sem1)

        def gather_start(c, buf):
            return pltpu.async_copy(
                w_hbm.at[idx_v.at[pl.ds(c * ch, ch)]], rows_v.at[buf], g_sem)

        def write_start(c, buf):
            return pltpu.async_copy(
                rows_v.at[buf], out_hbm.at[pl.ds(base + c * ch, ch)],
                w_sems[buf])

        g = gather_start(0, 0)
        wdescs = [None, None]
        for c in range(n_ch):
            buf = c % 2
            g.wait()
            if c + 1 < n_ch:
                nbuf = (c + 1) % 2
                if wdescs[nbuf] is not None:
                    wdescs[nbuf].wait()
                    wdescs[nbuf] = None
                g = gather_start(c + 1, nbuf)
            wdescs[buf] = write_start(c, buf)
        for wd in wdescs:
            if wd is not None:
                wd.wait()

    return sc_gather(x_flat, w_t)


def kernel(x, W_E):
    B, S = x.shape
    D, V = W_E.shape
    x_flat = x.reshape(B * S).astype(jnp.int32)
    w_t = W_E.T
    out = _sc_row_gather(x_flat, w_t)
    return out.reshape(B, S, D)


# speedup vs baseline: 13.2313x; 1.0244x over previous
import functools

import jax
import jax.numpy as jnp
from jax import lax
from jax.experimental import pallas as pl
from jax.experimental.pallas import tpu as pltpu
from jax.experimental.pallas import tpu_sc as plsc

_NUM_CORES = 2
_NUM_SUBCORES = 16
_NBUF = 4


def _sc_row_gather(x, w_t):
    V, D = w_t.shape
    B, S = x.shape
    T = B * S
    nw = _NUM_CORES * _NUM_SUBCORES
    b_per_w = T // nw
    w_per_row = S // b_per_w
    ch = 32
    n_ch = b_per_w // ch
    assert T % (8 * nw) == 0 and b_per_w % ch == 0 and S % b_per_w == 0

    mesh = plsc.VectorSubcoreMesh(
        core_axis_name="c", subcore_axis_name="s",
        num_cores=_NUM_CORES, num_subcores=_NUM_SUBCORES)

    @functools.partial(
        pl.kernel,
        out_type=jax.ShapeDtypeStruct((T, D), jnp.float32),
        mesh=mesh,
        scratch_types=[
            pltpu.VMEM((b_per_w,), jnp.int32),
            pltpu.VMEM((_NBUF, ch, D), jnp.float32),
            pltpu.SemaphoreType.DMA,
            pltpu.SemaphoreType.DMA,
            [pltpu.SemaphoreType.DMA] * _NBUF,
        ],
        compiler_params=pltpu.CompilerParams(needs_layout_passes=False),
    )
    def sc_gather(x_hbm, w_hbm, out_hbm, idx_v, rows_v, g_sem0, g_sem1,
                  w_sems):
        wid = lax.axis_index("s") * _NUM_CORES + lax.axis_index("c")
        base = wid * b_per_w
        pltpu.sync_copy(
            x_hbm.at[wid // w_per_row,
                     pl.ds((wid % w_per_row) * b_per_w, b_per_w)], idx_v)

        g_sems = (g_sem0, g_sem1)

        def gather_start(c):
            return pltpu.async_copy(
                w_hbm.at[idx_v.at[pl.ds(c * ch, ch)]],
                rows_v.at[c % _NBUF], g_sems[c % 2])

        def write_start(c):
            return pltpu.async_copy(
                rows_v.at[c % _NBUF], out_hbm.at[pl.ds(base + c * ch, ch)],
                w_sems[c % _NBUF])

        gathers = [gather_start(0), gather_start(1)] + [None] * (n_ch - 2)
        writes = [None] * n_ch
        for c in range(n_ch):
            gathers[c].wait()
            writes[c] = write_start(c)
            nxt = c + 2
            if nxt < n_ch:
                prev = nxt - _NBUF
                if prev >= 0:
                    writes[prev].wait()
                    writes[prev] = None
                gathers[nxt] = gather_start(nxt)
        for wd in writes:
            if wd is not None:
                wd.wait()

    return sc_gather(x, w_t)


def kernel(x, W_E):
    B, S = x.shape
    D, V = W_E.shape
    w_t = W_E.T
    out = _sc_row_gather(x.astype(jnp.int32), w_t)
    return out.reshape(B, S, D)
